# Initial kernel scaffold; baseline (speedup 1.0000x reference)
import jax
import jax.numpy as jnp
from jax.experimental import pallas as pl


def kernel(x_bmn, embed, gate_w, gate_b, ln_scale, ln_bias, h_init, head_w, head_b):
    raise NotImplementedError("write your pallas kernel here")


# speedup vs baseline: 24.3572x; 24.3572x over previous
import numpy as np
import jax
import jax.numpy as jnp
from jax.experimental import pallas as pl
from jax.experimental.pallas import tpu as pltpu

HID_ = 128
M_, N_, B_ = 28, 28, 32
T_ = M_ + N_ - 1
ROWS_ = M_ * B_


def _recurrence_kernel(xe_ref, wx_ref, wl_ref, wu_ref, b_ref, lns_ref,
                       lnb_ref, hinit_ref, hout_ref, H, C):
    t = pl.program_id(0)
    h = HID_

    @pl.when(t == 0)
    def _init():
        H[...] = jnp.broadcast_to(hinit_ref[...], (M_ + 1, B_, h))
        C[...] = jnp.zeros((M_ + 1, B_, h), jnp.float32)

    xe = xe_ref[0].reshape(ROWS_, 4 * h)
    h_left = H[1:].reshape(ROWS_, h)
    h_up = H[:M_].reshape(ROWS_, h)
    c_left = C[1:].reshape(ROWS_, h)
    c_up = C[:M_].reshape(ROWS_, h)

    g = (jnp.dot(xe, wx_ref[...], preferred_element_type=jnp.float32)
         + jnp.dot(h_left, wl_ref[...], preferred_element_type=jnp.float32)
         + jnp.dot(h_up, wu_ref[...], preferred_element_type=jnp.float32)
         + b_ref[...])
    fc = jax.nn.sigmoid(g[:, 0:h])
    fr = jax.nn.sigmoid(g[:, h:2 * h])
    ig = jax.nn.sigmoid(g[:, 2 * h:3 * h])
    og = jax.nn.sigmoid(g[:, 3 * h:4 * h])
    cg = jnp.tanh(g[:, 4 * h:5 * h])

    c_new = fc * c_left + fr * c_up + ig * cg
    mu = jnp.mean(c_new, axis=-1, keepdims=True)
    var = jnp.mean((c_new - mu) * (c_new - mu), axis=-1, keepdims=True)
    zn = (c_new - mu) * jax.lax.rsqrt(var + 1e-6) * lns_ref[...] + lnb_ref[...]
    s = og * jnp.tanh(zn)

    s3 = s.reshape(M_, B_, h)
    c3 = c_new.reshape(M_, B_, h)
    ii = jax.lax.broadcasted_iota(jnp.int32, (M_, 1, 1), 0)
    mask = (ii <= t) & (ii >= t - (N_ - 1))
    H[1:] = jnp.where(mask, s3, H[1:])
    C[1:] = jnp.where(mask, c3, C[1:])
    hout_ref[0] = s3


def _head_kernel(h_ref, w_ref, b_ref, o_ref):
    o_ref[...] = (jnp.dot(h_ref[...], w_ref[...],
                          preferred_element_type=jnp.float32) + b_ref[...])


def kernel(x_bmn, embed, gate_w, gate_b, ln_scale, ln_bias, h_init,
           head_w, head_b):
    b, m, n = x_bmn.shape
    h = h_init.shape[1]

    x_p = jnp.pad(x_bmn, ((0, 0), (1, 0), (1, 1)))
    E = jnp.take(embed, x_p, axis=0)
    A1 = E[:, 1:, 0:n]
    A2 = E[:, :m, 0:n]
    A3 = E[:, :m, 1:n + 1]
    A4 = E[:, :m, 2:n + 2]
    Xe = jnp.concatenate([A1, A2, A3, A4], axis=-1)
    tt, rr = np.meshgrid(np.arange(T_), np.arange(m), indexing='ij')
    jj = np.clip(tt - rr, 0, n - 1)
    xe_diag = jnp.transpose(Xe[:, rr, jj], (1, 2, 0, 3))

    wx = gate_w[:4 * h]
    wl = gate_w[4 * h:5 * h]
    wu = gate_w[5 * h:]
    b2 = gate_b.reshape(1, 5 * h)
    lns2 = ln_scale.reshape(1, h)
    lnb2 = ln_bias.reshape(1, h)

    h_diag = pl.pallas_call(
        _recurrence_kernel,
        grid=(T_,),
        in_specs=[
            pl.BlockSpec((1, M_, B_, 4 * h), lambda t: (t, 0, 0, 0)),
            pl.BlockSpec((4 * h, 5 * h), lambda t: (0, 0)),
            pl.BlockSpec((h, 5 * h), lambda t: (0, 0)),
            pl.BlockSpec((h, 5 * h), lambda t: (0, 0)),
            pl.BlockSpec((1, 5 * h), lambda t: (0, 0)),
            pl.BlockSpec((1, h), lambda t: (0, 0)),
            pl.BlockSpec((1, h), lambda t: (0, 0)),
            pl.BlockSpec((1, h), lambda t: (0, 0)),
        ],
        out_specs=pl.BlockSpec((1, M_, B_, h), lambda t: (t, 0, 0, 0)),
        out_shape=jax.ShapeDtypeStruct((T_, M_, B_, h), jnp.float32),
        scratch_shapes=[pltpu.VMEM((M_ + 1, B_, h), jnp.float32),
                        pltpu.VMEM((M_ + 1, B_, h), jnp.float32)],
        compiler_params=pltpu.CompilerParams(
            dimension_semantics=("arbitrary",)),
    )(xe_diag, wx, wl, wu, b2, lns2, lnb2, h_init)

    i2, j2 = np.meshgrid(np.arange(m), np.arange(n), indexing='ij')
    h_rows = h_diag[i2 + j2, i2].reshape(b * m * n, h)

    nblk = 8
    blk = (b * m * n) // nblk
    logits = pl.pallas_call(
        _head_kernel,
        grid=(nblk,),
        in_specs=[
            pl.BlockSpec((blk, h), lambda i: (i, 0)),
            pl.BlockSpec((h, 2 * h), lambda i: (0, 0)),
            pl.BlockSpec((1, 2 * h), lambda i: (0, 0)),
        ],
        out_specs=pl.BlockSpec((blk, 2 * h), lambda i: (i, 0)),
        out_shape=jax.ShapeDtypeStruct((b * m * n, 2 * h), jnp.float32),
    )(h_rows, head_w, head_b.reshape(1, 2 * h))

    return logits.reshape(b, m, n, 2 * h)


# speedup vs baseline: 30.2314x; 1.2412x over previous
import numpy as np
import jax
import jax.numpy as jnp
from jax.experimental import pallas as pl
from jax.experimental.pallas import tpu as pltpu

HID_ = 128
M_, N_, B_ = 28, 28, 32
T_ = M_ + N_ - 1
Q_ = T_ + 2
ROWS_ = M_ * B_


def _recurrence_kernel(e0_ref, e1_ref, e2_ref, wx_ref, wl_ref, wu_ref,
                       b_ref, lns_ref, lnb_ref, hinit_ref, hout_ref, H, C):
    t = pl.program_id(0)
    h = HID_

    @pl.when(t == 0)
    def _init():
        H[...] = jnp.broadcast_to(hinit_ref[...], (M_ + 1, B_, h))
        C[...] = jnp.zeros((M_ + 1, B_, h), jnp.float32)

    e0 = e0_ref[0]
    e1 = e1_ref[0]
    e2 = e2_ref[0]
    x1 = e1[1:].reshape(ROWS_, h)
    x2 = e0[:M_].reshape(ROWS_, h)
    x3 = e1[:M_].reshape(ROWS_, h)
    x4 = e2[:M_].reshape(ROWS_, h)
    h_left = H[1:].reshape(ROWS_, h)
    h_up = H[:M_].reshape(ROWS_, h)
    c_left = C[1:].reshape(ROWS_, h)
    c_up = C[:M_].reshape(ROWS_, h)

    f32 = jnp.float32
    g = (jnp.dot(x1, wx_ref[0:h], preferred_element_type=f32)
         + jnp.dot(x2, wx_ref[h:2 * h], preferred_element_type=f32)
         + jnp.dot(x3, wx_ref[2 * h:3 * h], preferred_element_type=f32)
         + jnp.dot(x4, wx_ref[3 * h:4 * h], preferred_element_type=f32)
         + jnp.dot(h_left, wl_ref[...], preferred_element_type=f32)
         + jnp.dot(h_up, wu_ref[...], preferred_element_type=f32)
         + b_ref[...])
    fc = jax.nn.sigmoid(g[:, 0:h])
    fr = jax.nn.sigmoid(g[:, h:2 * h])
    ig = jax.nn.sigmoid(g[:, 2 * h:3 * h])
    og = jax.nn.sigmoid(g[:, 3 * h:4 * h])
    cg = jnp.tanh(g[:, 4 * h:5 * h])

    c_new = fc * c_left + fr * c_up + ig * cg
    mu = jnp.mean(c_new, axis=-1, keepdims=True)
    var = jnp.mean((c_new - mu) * (c_new - mu), axis=-1, keepdims=True)
    zn = (c_new - mu) * jax.lax.rsqrt(var + 1e-6) * lns_ref[...] + lnb_ref[...]
    s = og * jnp.tanh(zn)

    s3 = s.reshape(M_, B_, h)
    c3 = c_new.reshape(M_, B_, h)
    ii = jax.lax.broadcasted_iota(jnp.int32, (M_, 1, 1), 0)
    mask = (ii <= t) & (ii >= t - (N_ - 1))
    H[1:] = jnp.where(mask, s3, H[1:])
    C[1:] = jnp.where(mask, c3, C[1:])
    hout_ref[0] = s3


def _head_kernel(h_ref, w_ref, b_ref, o_ref):
    o_ref[...] = (jnp.dot(h_ref[...], w_ref[...],
                          preferred_element_type=jnp.float32) + b_ref[...])


def kernel(x_bmn, embed, gate_w, gate_b, ln_scale, ln_bias, h_init,
           head_w, head_b):
    b, m, n = x_bmn.shape
    h = h_init.shape[1]

    x_p = jnp.pad(x_bmn, ((0, 0), (1, 0), (1, 1)))
    q_i = np.arange(Q_)[:, None]
    a_i = np.arange(m + 1)[None, :]
    c_i = np.clip(q_i - a_i, 0, n + 1)
    tok = jnp.transpose(x_p[:, a_i, c_i], (1, 2, 0))
    Ed = jnp.take(embed, tok, axis=0)

    wx = gate_w[:4 * h]
    wl = gate_w[4 * h:5 * h]
    wu = gate_w[5 * h:]
    b2 = gate_b.reshape(1, 5 * h)
    lns2 = ln_scale.reshape(1, h)
    lnb2 = ln_bias.reshape(1, h)

    ed_spec = lambda off: pl.BlockSpec(
        (1, m + 1, B_, h), lambda t, o=off: (t + o, 0, 0, 0))
    h_diag = pl.pallas_call(
        _recurrence_kernel,
        grid=(T_,),
        in_specs=[
            ed_spec(0), ed_spec(1), ed_spec(2),
            pl.BlockSpec((4 * h, 5 * h), lambda t: (0, 0)),
            pl.BlockSpec((h, 5 * h), lambda t: (0, 0)),
            pl.BlockSpec((h, 5 * h), lambda t: (0, 0)),
            pl.BlockSpec((1, 5 * h), lambda t: (0, 0)),
            pl.BlockSpec((1, h), lambda t: (0, 0)),
            pl.BlockSpec((1, h), lambda t: (0, 0)),
            pl.BlockSpec((1, h), lambda t: (0, 0)),
        ],
        out_specs=pl.BlockSpec((1, M_, B_, h), lambda t: (t, 0, 0, 0)),
        out_shape=jax.ShapeDtypeStruct((T_, M_, B_, h), jnp.float32),
        scratch_shapes=[pltpu.VMEM((M_ + 1, B_, h), jnp.float32),
                        pltpu.VMEM((M_ + 1, B_, h), jnp.float32)],
        compiler_params=pltpu.CompilerParams(
            dimension_semantics=("arbitrary",)),
    )(Ed, Ed, Ed, wx, wl, wu, b2, lns2, lnb2, h_init)

    i2, j2 = np.meshgrid(np.arange(m), np.arange(n), indexing='ij')
    h_rows = h_diag[i2 + j2, i2].reshape(b * m * n, h)

    nblk = 8
    blk = (b * m * n) // nblk
    logits = pl.pallas_call(
        _head_kernel,
        grid=(nblk,),
        in_specs=[
            pl.BlockSpec((blk, h), lambda i: (i, 0)),
            pl.BlockSpec((h, 2 * h), lambda i: (0, 0)),
            pl.BlockSpec((1, 2 * h), lambda i: (0, 0)),
        ],
        out_specs=pl.BlockSpec((blk, 2 * h), lambda i: (i, 0)),
        out_shape=jax.ShapeDtypeStruct((b * m * n, 2 * h), jnp.float32),
    )(h_rows, head_w, head_b.reshape(1, 2 * h))

    return logits.reshape(b, m, n, 2 * h)


# speedup vs baseline: 30.5710x; 1.0112x over previous
import numpy as np
import jax
import jax.numpy as jnp
from jax.experimental import pallas as pl
from jax.experimental.pallas import tpu as pltpu

HID_ = 128
M_, N_, B_ = 28, 28, 32
T_ = M_ + N_ - 1
Q_ = T_ + 2
ROWS_ = M_ * B_


def _recurrence_kernel(e0_ref, e1_ref, e2_ref, wx_ref, wl_ref, wu_ref,
                       b_ref, lns_ref, lnb_ref, hinit_ref, hout_ref, H, C):
    t = pl.program_id(0)
    h = HID_

    @pl.when(t == 0)
    def _init():
        H[...] = jnp.broadcast_to(hinit_ref[...], (M_ + 1, B_, h))
        C[...] = jnp.zeros((M_ + 1, B_, h), jnp.float32)

    e0 = e0_ref[0]
    e1 = e1_ref[0]
    e2 = e2_ref[0]
    x1 = e1[1:].reshape(ROWS_, h)
    x2 = e0[:M_].reshape(ROWS_, h)
    x3 = e1[:M_].reshape(ROWS_, h)
    x4 = e2[:M_].reshape(ROWS_, h)
    h_left = H[1:].reshape(ROWS_, h)
    h_up = H[:M_].reshape(ROWS_, h)
    c_left = C[1:].reshape(ROWS_, h)
    c_up = C[:M_].reshape(ROWS_, h)

    f32 = jnp.float32
    g = (jnp.dot(x1, wx_ref[0:h], preferred_element_type=f32)
         + jnp.dot(x2, wx_ref[h:2 * h], preferred_element_type=f32)
         + jnp.dot(x3, wx_ref[2 * h:3 * h], preferred_element_type=f32)
         + jnp.dot(x4, wx_ref[3 * h:4 * h], preferred_element_type=f32)
         + jnp.dot(h_left, wl_ref[...], preferred_element_type=f32)
         + jnp.dot(h_up, wu_ref[...], preferred_element_type=f32)
         + b_ref[...])
    fc = jax.nn.sigmoid(g[:, 0:h])
    fr = jax.nn.sigmoid(g[:, h:2 * h])
    ig = jax.nn.sigmoid(g[:, 2 * h:3 * h])
    og = jax.nn.sigmoid(g[:, 3 * h:4 * h])
    cg = jnp.tanh(g[:, 4 * h:5 * h])

    c_new = fc * c_left + fr * c_up + ig * cg
    mu = jnp.mean(c_new, axis=-1, keepdims=True)
    var = jnp.mean((c_new - mu) * (c_new - mu), axis=-1, keepdims=True)
    zn = (c_new - mu) * jax.lax.rsqrt(var + 1e-6) * lns_ref[...] + lnb_ref[...]
    s = og * jnp.tanh(zn)

    s3 = s.reshape(M_, B_, h)
    c3 = c_new.reshape(M_, B_, h)
    ii = jax.lax.broadcasted_iota(jnp.int32, (M_, 1, 1), 0)
    mask = (ii <= t) & (ii >= t - (N_ - 1))
    H[1:] = jnp.where(mask, s3, H[1:])
    C[1:] = jnp.where(mask, c3, C[1:])
    hout_ref[0] = s3


def _head_kernel(h_ref, w_ref, b_ref, o_ref):
    o_ref[...] = (jnp.dot(h_ref[...], w_ref[...],
                          preferred_element_type=jnp.float32) + b_ref[...])


def kernel(x_bmn, embed, gate_w, gate_b, ln_scale, ln_bias, h_init,
           head_w, head_b):
    b, m, n = x_bmn.shape
    h = h_init.shape[1]

    x_p = jnp.pad(x_bmn, ((0, 0), (1, 0), (1, 1)))
    q_i = np.arange(Q_)[:, None]
    a_i = np.arange(m + 1)[None, :]
    c_i = np.clip(q_i - a_i, 0, n + 1)
    tok = jnp.transpose(x_p[:, a_i, c_i], (1, 2, 0))
    Ed = jnp.take(embed.astype(jnp.bfloat16), tok, axis=0)

    wx = gate_w[:4 * h].astype(jnp.bfloat16)
    wl = gate_w[4 * h:5 * h]
    wu = gate_w[5 * h:]
    b2 = gate_b.reshape(1, 5 * h)
    lns2 = ln_scale.reshape(1, h)
    lnb2 = ln_bias.reshape(1, h)

    ed_spec = lambda off: pl.BlockSpec(
        (1, m + 1, B_, h), lambda t, o=off: (t + o, 0, 0, 0))
    h_diag = pl.pallas_call(
        _recurrence_kernel,
        grid=(T_,),
        in_specs=[
            ed_spec(0), ed_spec(1), ed_spec(2),
            pl.BlockSpec((4 * h, 5 * h), lambda t: (0, 0)),
            pl.BlockSpec((h, 5 * h), lambda t: (0, 0)),
            pl.BlockSpec((h, 5 * h), lambda t: (0, 0)),
            pl.BlockSpec((1, 5 * h), lambda t: (0, 0)),
            pl.BlockSpec((1, h), lambda t: (0, 0)),
            pl.BlockSpec((1, h), lambda t: (0, 0)),
            pl.BlockSpec((1, h), lambda t: (0, 0)),
        ],
        out_specs=pl.BlockSpec((1, M_, B_, h), lambda t: (t, 0, 0, 0)),
        out_shape=jax.ShapeDtypeStruct((T_, M_, B_, h), jnp.float32),
        scratch_shapes=[pltpu.VMEM((M_ + 1, B_, h), jnp.float32),
                        pltpu.VMEM((M_ + 1, B_, h), jnp.float32)],
        compiler_params=pltpu.CompilerParams(
            dimension_semantics=("arbitrary",)),
    )(Ed, Ed, Ed, wx, wl, wu, b2, lns2, lnb2, h_init)

    i2, j2 = np.meshgrid(np.arange(m), np.arange(n), indexing='ij')
    h_rows = h_diag[i2 + j2, i2].reshape(b * m * n, h)

    nblk = 8
    blk = (b * m * n) // nblk
    logits = pl.pallas_call(
        _head_kernel,
        grid=(nblk,),
        in_specs=[
            pl.BlockSpec((blk, h), lambda i: (i, 0)),
            pl.BlockSpec((h, 2 * h), lambda i: (0, 0)),
            pl.BlockSpec((1, 2 * h), lambda i: (0, 0)),
        ],
        out_specs=pl.BlockSpec((blk, 2 * h), lambda i: (i, 0)),
        out_shape=jax.ShapeDtypeStruct((b * m * n, 2 * h), jnp.float32),
    )(h_rows, head_w, head_b.reshape(1, 2 * h))

    return logits.reshape(b, m, n, 2 * h)
